# baseline (speedup 1.0000x reference)
import functools

import jax
import jax.numpy as jnp
from jax import lax
from jax.experimental import pallas as pl
from jax.experimental.pallas import tpu as pltpu
from jax.experimental.pallas import tpu_sc as plsc


def _sc_gather(t, emb):
    T = t.shape[0]
    V, D = emb.shape
    info = plsc.get_sparse_core_info()
    nc, ns = info.num_cores, info.num_subcores
    nw = nc * ns
    rows_per_w = T // nw

    mesh = plsc.VectorSubcoreMesh(core_axis_name="c", subcore_axis_name="s")

    @functools.partial(
        pl.kernel,
        mesh=mesh,
        out_type=jax.ShapeDtypeStruct((T, D), jnp.float32),
        scratch_types=[
            pltpu.VMEM((rows_per_w,), jnp.int32),
            pltpu.VMEM((rows_per_w, D), jnp.float32),
            pltpu.SemaphoreType.DMA,
        ],
    )
    def gather_kernel(t_hbm, emb_hbm, out_hbm, idx_v, rows_v, sem):
        wid = lax.axis_index("s") * nc + lax.axis_index("c")
        base = wid * rows_per_w
        pltpu.sync_copy(t_hbm.at[pl.ds(base, rows_per_w)], idx_v)
        pltpu.async_copy(emb_hbm.at[idx_v], rows_v, sem).wait()
        pltpu.sync_copy(rows_v, out_hbm.at[pl.ds(base, rows_per_w)])

    return gather_kernel(t, emb)


_TR = 64


def _tc_body(x_ref, e_ref, w16_ref, b_ref, out_ref):
    rows = e_ref[...].astype(jnp.bfloat16)
    te = lax.dot_general(
        rows, w16_ref[...],
        dimension_numbers=(((1,), (1,)), ((), ())),
        preferred_element_type=jnp.float32,
    ) + b_ref[...]
    out_ref[...] = x_ref[...] + te[None]


def _proj_add(x3, e, W16, b, interpret=False):
    BH, T, D = x3.shape
    nt = T // _TR
    b2 = b.reshape(1, D)
    return pl.pallas_call(
        _tc_body,
        grid=(nt,),
        in_specs=[
            pl.BlockSpec((BH, _TR, D), lambda j: (0, j, 0)),
            pl.BlockSpec((_TR, D), lambda j: (j, 0)),
            pl.BlockSpec((D, D), lambda j: (0, 0)),
            pl.BlockSpec((1, D), lambda j: (0, 0)),
        ],
        out_specs=pl.BlockSpec((BH, _TR, D), lambda j: (0, j, 0)),
        out_shape=jax.ShapeDtypeStruct((BH, T, D), jnp.float32),
        interpret=interpret,
    )(x3, e, W16, b2)


def kernel(x, t, emb, W, b):
    B, H, T, D = x.shape
    e = _sc_gather(t, emb)
    x3 = x.reshape(B * H, T, D)
    out3 = _proj_add(x3, e, W.astype(jnp.bfloat16), b)
    return out3.reshape(B, H, T, D)


# speedup vs baseline: 1.0092x; 1.0092x over previous
import functools

import jax
import jax.numpy as jnp
from jax import lax
from jax.experimental import pallas as pl
from jax.experimental.pallas import tpu as pltpu
from jax.experimental.pallas import tpu_sc as plsc


def _sc_gather(t, emb):
    T = t.shape[0]
    V, D = emb.shape
    info = plsc.get_sparse_core_info()
    nc, ns = info.num_cores, info.num_subcores
    nw = nc * ns
    rows_per_w = T // nw

    mesh = plsc.VectorSubcoreMesh(core_axis_name="c", subcore_axis_name="s")

    @functools.partial(
        pl.kernel,
        mesh=mesh,
        out_type=jax.ShapeDtypeStruct((T, D), jnp.float32),
        scratch_types=[
            pltpu.VMEM((rows_per_w,), jnp.int32),
            pltpu.VMEM((rows_per_w, D), jnp.float32),
            pltpu.SemaphoreType.DMA,
        ],
    )
    def gather_kernel(t_hbm, emb_hbm, out_hbm, idx_v, rows_v, sem):
        wid = lax.axis_index("s") * nc + lax.axis_index("c")
        base = wid * rows_per_w
        pltpu.sync_copy(t_hbm.at[pl.ds(base, rows_per_w)], idx_v)
        pltpu.async_copy(emb_hbm.at[idx_v], rows_v, sem).wait()
        pltpu.sync_copy(rows_v, out_hbm.at[pl.ds(base, rows_per_w)])

    return gather_kernel(t, emb)


_TR = 128
_BG = 16


def _tc_body(x_ref, e_ref, w16_ref, b_ref, out_ref, te_ref):
    i = pl.program_id(0)
    j = pl.program_id(1)

    @pl.when(i == 0)
    def _():
        rows = e_ref[...].astype(jnp.bfloat16)
        te_ref[pl.ds(j * _TR, _TR), :] = lax.dot_general(
            rows, w16_ref[...],
            dimension_numbers=(((1,), (1,)), ((), ())),
            preferred_element_type=jnp.float32,
        ) + b_ref[...]

    out_ref[...] = x_ref[...] + te_ref[pl.ds(j * _TR, _TR), :][None]


def _proj_add(x3, e, W16, b, interpret=False):
    BH, T, D = x3.shape
    nt = T // _TR
    b2 = b.reshape(1, D)
    return pl.pallas_call(
        _tc_body,
        grid=(BH // _BG, nt),
        in_specs=[
            pl.BlockSpec((_BG, _TR, D), lambda i, j: (i, j, 0)),
            pl.BlockSpec((_TR, D),
                         lambda i, j: (jnp.where(i == 0, j, nt - 1), 0)),
            pl.BlockSpec((D, D), lambda i, j: (0, 0)),
            pl.BlockSpec((1, D), lambda i, j: (0, 0)),
        ],
        out_specs=pl.BlockSpec((_BG, _TR, D), lambda i, j: (i, j, 0)),
        out_shape=jax.ShapeDtypeStruct((BH, T, D), jnp.float32),
        scratch_shapes=[
            pltpu.VMEM((T, D), jnp.float32),
        ],
        interpret=interpret,
    )(x3, e, W16, b2)


def kernel(x, t, emb, W, b):
    B, H, T, D = x.shape
    e = _sc_gather(t, emb)
    x3 = x.reshape(B * H, T, D)
    out3 = _proj_add(x3, e, W.astype(jnp.bfloat16), b)
    return out3.reshape(B, H, T, D)
